# baseline (speedup 1.0000x reference)
import jax
import jax.numpy as jnp
from jax import lax
from jax.experimental import pallas as pl
from jax.experimental.pallas import tpu as pltpu
from jax.experimental.pallas import tpu_sc as plsc

_NC = 2
_NS = 16
_NW = _NC * _NS
_CHUNK = 128
_NCH = 4
_RPW = _CHUNK * _NCH
_K = 4
_ES = 32
_L = 16


def _fused_body(ih_hbm, ir_hbm, it_hbm, ent_hbm, rel_hbm, w_hbm, b_hbm,
                pos_out, neg_out, y_out, att_out,
                idxh_v, idxr_v, idxt_v,
                h0, h1, t0, t1, r0, r1,
                norm_v, att_v, y_v, w_v, b_v,
                sem_h0, sem_h1, sem_t0, sem_t1, sem_r0, sem_r1):
    wid = lax.axis_index("s") * _NC + lax.axis_index("c")
    base_row = wid * _RPW
    len_pos = (_NW * _RPW) // 4
    pltpu.sync_copy(w_hbm, w_v)
    pltpu.sync_copy(b_hbm, b_v)
    iota = lax.iota(jnp.int32, _L)

    pltpu.sync_copy(ih_hbm.at[pl.ds(base_row, _RPW)], idxh_v)
    pltpu.sync_copy(ir_hbm.at[pl.ds(base_row, _RPW)], idxr_v)
    pltpu.sync_copy(it_hbm.at[pl.ds(base_row, _RPW)], idxt_v)

    h_bufs, t_bufs, r_bufs = (h0, h1), (t0, t1), (r0, r1)
    sem_h, sem_t, sem_r = (sem_h0, sem_h1), (sem_t0, sem_t1), (sem_r0, sem_r1)

    def issue(ch):
        p = ch % 2
        cph = pltpu.async_copy(
            ent_hbm.at[idxh_v.at[pl.ds(ch * _CHUNK, _CHUNK)]], h_bufs[p], sem_h[p])
        cpt = pltpu.async_copy(
            ent_hbm.at[idxt_v.at[pl.ds(ch * _CHUNK, _CHUNK)]], t_bufs[p], sem_t[p])

        def rel_issue(g, _):
            vec = idxr_v[pl.ds(ch * _CHUNK + g * _L, _L)]
            for j in range(_L):
                pltpu.async_copy(rel_hbm.at[vec[j]],
                                 r_bufs[p].at[g * _L + j], sem_r[p])
            return 0

        lax.fori_loop(0, _CHUNK // _L, rel_issue, 0)
        return cph, cpt

    def wait(ch, cph, cpt):
        p = ch % 2
        cph.wait()
        cpt.wait()
        pltpu.make_async_copy(rel_hbm.at[pl.ds(0, _CHUNK)], r_bufs[p],
                              sem_r[p]).wait()

    wvecs = [plsc.load_gather(w_v, [jnp.zeros((_L,), jnp.int32), iota + j * _L])
             for j in range(6)]
    bias = plsc.load_gather(b_v, [jnp.zeros((_L,), jnp.int32)])
    zero = jnp.zeros((_L,), jnp.float32)
    m15 = iota == 15
    m4 = iota < _K
    neg1 = jnp.full((_L,), -1.0, jnp.float32)
    for g in range(_RPW // _L):
        y_v[pl.ds(g * _L, _L)] = neg1

    pending = issue(0)

    for ch in range(_NCH):
        p = ch % 2
        wait(ch, *pending)
        if ch + 1 < _NCH:
            pending = issue(ch + 1)
        hb, tb, rb = h_bufs[p], t_bufs[p], r_bufs[p]

        @plsc.parallel_loop(0, _CHUNK, unroll=4)
        def row_body(i):
            ri = jnp.full((_L,), i, jnp.int32)
            hv = [plsc.load_gather(hb, [ri, iota + j * _L]) for j in range(8)]
            tv = [plsc.load_gather(tb, [ri, iota + j * _L]) for j in range(8)]
            rv = [plsc.load_gather(rb, [ri, iota]),
                  plsc.load_gather(rb, [ri, iota + _L])]
            difs = [hv[j] - tv[j] for j in range(8)]
            sr = jnp.sum(rv[0] * wvecs[2] + rv[1] * wvecs[3]) + bias
            ts = []
            for k in range(_K):
                pk = (hv[2 * k] * wvecs[0] + hv[2 * k + 1] * wvecs[1]
                      + tv[2 * k] * wvecs[4] + tv[2 * k + 1] * wvecs[5])
                ts.append(jnp.maximum(zero + jnp.sum(pk) + sr, 0.0))
            m = jnp.maximum(jnp.maximum(ts[0], ts[1]), jnp.maximum(ts[2], ts[3]))
            es = [jnp.exp(tk - m) for tk in ts]
            inv = 1.0 / (es[0] + es[1] + es[2] + es[3])
            att = [e * inv for e in es]
            x0 = rv[0] + att[0] * difs[0] + att[1] * difs[2] \
                + att[2] * difs[4] + att[3] * difs[6]
            x1 = rv[1] + att[0] * difs[1] + att[1] * difs[3] \
                + att[2] * difs[5] + att[3] * difs[7]
            nrm = jnp.cumsum(jnp.abs(x0) + jnp.abs(x1))
            pos = ch * _CHUNK + i
            plsc.store_scatter(norm_v, [jnp.full((_L,), pos, jnp.int32)], nrm,
                               mask=m15)
            av = jnp.where(iota == 0, att[0],
                           jnp.where(iota == 1, att[1],
                                     jnp.where(iota == 2, att[2], att[3])))
            plsc.store_scatter(att_v, [jnp.full((_L,), pos * _K, jnp.int32) + iota],
                               av, mask=m4)

        del row_body

    pltpu.sync_copy(att_v, att_out.at[pl.ds(base_row * _K, _RPW * _K)])

    @pl.when(base_row < len_pos)
    def _write_pos():
        for j in range(3):
            pltpu.sync_copy(norm_v, pos_out.at[pl.ds(base_row + j * len_pos, _RPW)])

    @pl.when(base_row >= len_pos)
    def _write_neg():
        pltpu.sync_copy(norm_v, neg_out.at[pl.ds(base_row - len_pos, _RPW)])
        pltpu.sync_copy(y_v, y_out.at[pl.ds(base_row - len_pos, _RPW)])


def _sc_fused(ih, ir, it, entity_emb, relation_emb, fc1_w, fc1_b):
    b = _NW * _RPW
    dr = relation_emb.shape[1]
    mesh = plsc.VectorSubcoreMesh(core_axis_name="c", subcore_axis_name="s",
                                  num_cores=_NC, num_subcores=_NS)
    return pl.kernel(
        _fused_body,
        out_type=(
            jax.ShapeDtypeStruct((3 * (b // 4),), jnp.float32),
            jax.ShapeDtypeStruct((3 * (b // 4),), jnp.float32),
            jax.ShapeDtypeStruct((3 * (b // 4),), jnp.float32),
            jax.ShapeDtypeStruct((b * _K,), jnp.float32),
        ),
        mesh=mesh,
        scratch_types=[
            pltpu.VMEM((_RPW,), jnp.int32),
            pltpu.VMEM((_RPW,), jnp.int32),
            pltpu.VMEM((_RPW,), jnp.int32),
            pltpu.VMEM((_CHUNK, 128), jnp.float32),
            pltpu.VMEM((_CHUNK, 128), jnp.float32),
            pltpu.VMEM((_CHUNK, 128), jnp.float32),
            pltpu.VMEM((_CHUNK, 128), jnp.float32),
            pltpu.VMEM((_CHUNK, _ES), jnp.float32),
            pltpu.VMEM((_CHUNK, _ES), jnp.float32),
            pltpu.VMEM((_RPW,), jnp.float32),
            pltpu.VMEM((_RPW * _K,), jnp.float32),
            pltpu.VMEM((_RPW,), jnp.float32),
            pltpu.VMEM((1, 3 * _ES), jnp.float32),
            pltpu.VMEM((1,), jnp.float32),
            pltpu.SemaphoreType.DMA,
            pltpu.SemaphoreType.DMA,
            pltpu.SemaphoreType.DMA,
            pltpu.SemaphoreType.DMA,
            pltpu.SemaphoreType.DMA,
            pltpu.SemaphoreType.DMA,
        ],
        compiler_params=pltpu.CompilerParams(needs_layout_passes=False),
    )(ih, ir, it, entity_emb, relation_emb, fc1_w, fc1_b)


def kernel(batch_inputs, entity_emb, relation_emb, fc1_w, fc1_b):
    ih = batch_inputs[:, 0]
    ir = batch_inputs[:, 1]
    it = batch_inputs[:, 2]
    pos_norm, neg_norm, y, att_pad = _sc_fused(
        ih, ir, it, entity_emb, relation_emb, fc1_w, fc1_b)
    att = att_pad.reshape(batch_inputs.shape[0], _K)
    return (pos_norm, neg_norm, y, att)


# speedup vs baseline: 1.1361x; 1.1361x over previous
import jax
import jax.numpy as jnp
from jax import lax
from jax.experimental import pallas as pl
from jax.experimental.pallas import tpu as pltpu
from jax.experimental.pallas import tpu_sc as plsc

_NC = 2
_NS = 16
_NW = _NC * _NS
_CHUNK = 128
_NCH = 4
_RPW = _CHUNK * _NCH
_K = 4
_ES = 32
_L = 16


def _fused_body(ih_hbm, ir_hbm, it_hbm, ent_hbm, rel_hbm, w_hbm, b_hbm,
                pos_out, neg_out, y_out, a0_out, a1_out, a2_out, a3_out,
                idxh_v, idxr_v, idxt_v,
                h0, h1, t0, t1, r0, r1,
                norm_v, att_v, y_v, w_v, b_v,
                sem_h0, sem_h1, sem_t0, sem_t1, sem_r0, sem_r1):
    wid = lax.axis_index("s") * _NC + lax.axis_index("c")
    base_row = wid * _RPW
    len_pos = (_NW * _RPW) // 4
    pltpu.sync_copy(w_hbm, w_v)
    pltpu.sync_copy(b_hbm, b_v)
    iota = lax.iota(jnp.int32, _L)

    pltpu.sync_copy(ih_hbm.at[pl.ds(base_row, _RPW)], idxh_v)
    pltpu.sync_copy(ir_hbm.at[pl.ds(base_row, _RPW)], idxr_v)
    pltpu.sync_copy(it_hbm.at[pl.ds(base_row, _RPW)], idxt_v)

    h_bufs, t_bufs, r_bufs = (h0, h1), (t0, t1), (r0, r1)
    sem_h, sem_t, sem_r = (sem_h0, sem_h1), (sem_t0, sem_t1), (sem_r0, sem_r1)

    def issue(ch):
        p = ch % 2
        cph = pltpu.async_copy(
            ent_hbm.at[idxh_v.at[pl.ds(ch * _CHUNK, _CHUNK)]], h_bufs[p], sem_h[p])
        cpt = pltpu.async_copy(
            ent_hbm.at[idxt_v.at[pl.ds(ch * _CHUNK, _CHUNK)]], t_bufs[p], sem_t[p])

        def rel_issue(g, _):
            vec = idxr_v[pl.ds(ch * _CHUNK + g * _L, _L)]
            for j in range(_L):
                pltpu.async_copy(rel_hbm.at[vec[j]],
                                 r_bufs[p].at[g * _L + j], sem_r[p])
            return 0

        lax.fori_loop(0, _CHUNK // _L, rel_issue, 0)
        return cph, cpt

    def wait(ch, cph, cpt):
        p = ch % 2
        cph.wait()
        cpt.wait()
        pltpu.make_async_copy(rel_hbm.at[pl.ds(0, _CHUNK)], r_bufs[p],
                              sem_r[p]).wait()

    wvecs = [plsc.load_gather(w_v, [jnp.zeros((_L,), jnp.int32), iota + j * _L])
             for j in range(6)]
    bias = plsc.load_gather(b_v, [jnp.zeros((_L,), jnp.int32)])
    zero = jnp.zeros((_L,), jnp.float32)
    m15 = iota == 15
    m0 = iota == 0
    neg1 = jnp.full((_L,), -1.0, jnp.float32)
    for g in range(_RPW // _L):
        y_v[pl.ds(g * _L, _L)] = neg1

    pending = issue(0)

    for ch in range(_NCH):
        p = ch % 2
        wait(ch, *pending)
        if ch + 1 < _NCH:
            pending = issue(ch + 1)
        hb, tb, rb = h_bufs[p], t_bufs[p], r_bufs[p]

        @plsc.parallel_loop(0, _CHUNK, unroll=2)
        def row_body(i):
            ri = jnp.full((_L,), i, jnp.int32)
            hv = [plsc.load_gather(hb, [ri, iota + j * _L]) for j in range(8)]
            tv = [plsc.load_gather(tb, [ri, iota + j * _L]) for j in range(8)]
            rv = [plsc.load_gather(rb, [ri, iota]),
                  plsc.load_gather(rb, [ri, iota + _L])]
            difs = [hv[j] - tv[j] for j in range(8)]
            sr = jnp.sum(rv[0] * wvecs[2] + rv[1] * wvecs[3]) + bias
            ts = []
            for k in range(_K):
                pk = (hv[2 * k] * wvecs[0] + hv[2 * k + 1] * wvecs[1]
                      + tv[2 * k] * wvecs[4] + tv[2 * k + 1] * wvecs[5])
                ts.append(jnp.maximum(zero + jnp.sum(pk) + sr, 0.0))
            m = jnp.maximum(jnp.maximum(ts[0], ts[1]), jnp.maximum(ts[2], ts[3]))
            es = [jnp.exp(tk - m) for tk in ts]
            inv = 1.0 / (es[0] + es[1] + es[2] + es[3])
            att = [e * inv for e in es]
            x0 = rv[0] + att[0] * difs[0] + att[1] * difs[2] \
                + att[2] * difs[4] + att[3] * difs[6]
            x1 = rv[1] + att[0] * difs[1] + att[1] * difs[3] \
                + att[2] * difs[5] + att[3] * difs[7]
            nrm = jnp.cumsum(jnp.abs(x0) + jnp.abs(x1))
            pos = ch * _CHUNK + i
            plsc.store_scatter(norm_v, [jnp.full((_L,), pos, jnp.int32)], nrm,
                               mask=m15)
            for k in range(_K):
                plsc.store_scatter(
                    att_v, [jnp.full((_L,), k * _RPW + pos, jnp.int32)],
                    att[k], mask=m0)

        del row_body

    for k, ao in enumerate((a0_out, a1_out, a2_out, a3_out)):
        pltpu.sync_copy(att_v.at[pl.ds(k * _RPW, _RPW)],
                        ao.at[pl.ds(base_row, _RPW)])

    @pl.when(base_row < len_pos)
    def _write_pos():
        for j in range(3):
            pltpu.sync_copy(norm_v, pos_out.at[pl.ds(base_row + j * len_pos, _RPW)])

    @pl.when(base_row >= len_pos)
    def _write_neg():
        pltpu.sync_copy(norm_v, neg_out.at[pl.ds(base_row - len_pos, _RPW)])
        pltpu.sync_copy(y_v, y_out.at[pl.ds(base_row - len_pos, _RPW)])


def _sc_fused(ih, ir, it, entity_emb, relation_emb, fc1_w, fc1_b):
    b = _NW * _RPW
    dr = relation_emb.shape[1]
    mesh = plsc.VectorSubcoreMesh(core_axis_name="c", subcore_axis_name="s",
                                  num_cores=_NC, num_subcores=_NS)
    return pl.kernel(
        _fused_body,
        out_type=(
            jax.ShapeDtypeStruct((3 * (b // 4),), jnp.float32),
            jax.ShapeDtypeStruct((3 * (b // 4),), jnp.float32),
            jax.ShapeDtypeStruct((3 * (b // 4),), jnp.float32),
            jax.ShapeDtypeStruct((b,), jnp.float32),
            jax.ShapeDtypeStruct((b,), jnp.float32),
            jax.ShapeDtypeStruct((b,), jnp.float32),
            jax.ShapeDtypeStruct((b,), jnp.float32),
        ),
        mesh=mesh,
        scratch_types=[
            pltpu.VMEM((_RPW,), jnp.int32),
            pltpu.VMEM((_RPW,), jnp.int32),
            pltpu.VMEM((_RPW,), jnp.int32),
            pltpu.VMEM((_CHUNK, 128), jnp.float32),
            pltpu.VMEM((_CHUNK, 128), jnp.float32),
            pltpu.VMEM((_CHUNK, 128), jnp.float32),
            pltpu.VMEM((_CHUNK, 128), jnp.float32),
            pltpu.VMEM((_CHUNK, _ES), jnp.float32),
            pltpu.VMEM((_CHUNK, _ES), jnp.float32),
            pltpu.VMEM((_RPW,), jnp.float32),
            pltpu.VMEM((_RPW * _K,), jnp.float32),
            pltpu.VMEM((_RPW,), jnp.float32),
            pltpu.VMEM((1, 3 * _ES), jnp.float32),
            pltpu.VMEM((1,), jnp.float32),
            pltpu.SemaphoreType.DMA,
            pltpu.SemaphoreType.DMA,
            pltpu.SemaphoreType.DMA,
            pltpu.SemaphoreType.DMA,
            pltpu.SemaphoreType.DMA,
            pltpu.SemaphoreType.DMA,
        ],
        compiler_params=pltpu.CompilerParams(needs_layout_passes=False),
    )(ih, ir, it, entity_emb, relation_emb, fc1_w, fc1_b)


def kernel(batch_inputs, entity_emb, relation_emb, fc1_w, fc1_b):
    ih = batch_inputs[:, 0]
    ir = batch_inputs[:, 1]
    it = batch_inputs[:, 2]
    pos_norm, neg_norm, y, a0, a1, a2, a3 = _sc_fused(
        ih, ir, it, entity_emb, relation_emb, fc1_w, fc1_b)
    att = jnp.stack([a0, a1, a2, a3], axis=1)
    return (pos_norm, neg_norm, y, att)


# speedup vs baseline: 1.2763x; 1.1234x over previous
import jax
import jax.numpy as jnp
from jax import lax
from jax.experimental import pallas as pl
from jax.experimental.pallas import tpu as pltpu
from jax.experimental.pallas import tpu_sc as plsc

_NC = 2
_NS = 16
_NW = _NC * _NS
_CHUNK = 128
_NCH = 4
_RPW = _CHUNK * _NCH
_K = 4
_ES = 32
_L = 16


def _fused_body(ih_hbm, ir_hbm, it_hbm, ent_hbm, rel_hbm, w_hbm, b_hbm,
                pos_out, neg_out, y_out, a0_out, a1_out, a2_out, a3_out,
                idxh_v, idxr_v, idxt_v,
                h0, h1, t0, t1, r0, r1,
                norm_v, att_v, y_v, w_v, b_v,
                sem_h0, sem_h1, sem_t0, sem_t1, sem_r0, sem_r1):
    wid = lax.axis_index("s") * _NC + lax.axis_index("c")
    base_row = wid * _RPW
    len_pos = (_NW * _RPW) // 4
    pltpu.sync_copy(w_hbm, w_v)
    pltpu.sync_copy(b_hbm, b_v)
    iota = lax.iota(jnp.int32, _L)

    pltpu.sync_copy(ih_hbm.at[pl.ds(base_row, _RPW)], idxh_v)
    pltpu.sync_copy(ir_hbm.at[pl.ds(base_row, _RPW)], idxr_v)
    pltpu.sync_copy(it_hbm.at[pl.ds(base_row, _RPW)], idxt_v)

    h_bufs, t_bufs, r_bufs = (h0, h1), (t0, t1), (r0, r1)
    sem_h, sem_t, sem_r = (sem_h0, sem_h1), (sem_t0, sem_t1), (sem_r0, sem_r1)

    def issue(ch):
        p = ch % 2
        cph = pltpu.async_copy(
            ent_hbm.at[idxh_v.at[pl.ds(ch * _CHUNK, _CHUNK)]], h_bufs[p], sem_h[p])
        cpt = pltpu.async_copy(
            ent_hbm.at[idxt_v.at[pl.ds(ch * _CHUNK, _CHUNK)]], t_bufs[p], sem_t[p])

        def rel_issue(g, _):
            vec = idxr_v[pl.ds(ch * _CHUNK + g * _L, _L)]
            for j in range(_L):
                pltpu.async_copy(rel_hbm.at[vec[j]],
                                 r_bufs[p].at[g * _L + j], sem_r[p])
            return 0

        lax.fori_loop(0, _CHUNK // _L, rel_issue, 0)
        return cph, cpt

    def wait(ch, cph, cpt):
        p = ch % 2
        cph.wait()
        cpt.wait()
        pltpu.make_async_copy(rel_hbm.at[pl.ds(0, _CHUNK)], r_bufs[p],
                              sem_r[p]).wait()

    wvecs = [plsc.load_gather(w_v, [jnp.zeros((_L,), jnp.int32), iota + j * _L])
             for j in range(6)]
    bias = plsc.load_gather(b_v, [jnp.zeros((_L,), jnp.int32)])
    zero = jnp.zeros((_L,), jnp.float32)
    m15 = iota == 15
    m0 = iota == 0
    neg1 = jnp.full((_L,), -1.0, jnp.float32)
    for g in range(_RPW // _L):
        y_v[pl.ds(g * _L, _L)] = neg1

    pending = issue(0)

    for ch in range(_NCH):
        p = ch % 2
        wait(ch, *pending)
        if ch + 1 < _NCH:
            pending = issue(ch + 1)
        hb, tb, rb = h_bufs[p], t_bufs[p], r_bufs[p]

        @plsc.parallel_loop(0, _CHUNK, unroll=2)
        def row_body(i):
            hv = [hb[i, pl.ds(j * _L, _L)] for j in range(8)]
            tv = [tb[i, pl.ds(j * _L, _L)] for j in range(8)]
            rv = [rb[i, pl.ds(0, _L)], rb[i, pl.ds(_L, _L)]]
            difs = [hv[j] - tv[j] for j in range(8)]
            sr = jnp.sum(rv[0] * wvecs[2] + rv[1] * wvecs[3]) + bias
            ts = []
            for k in range(_K):
                pk = (hv[2 * k] * wvecs[0] + hv[2 * k + 1] * wvecs[1]
                      + tv[2 * k] * wvecs[4] + tv[2 * k + 1] * wvecs[5])
                ts.append(jnp.maximum(zero + jnp.sum(pk) + sr, 0.0))
            m = jnp.maximum(jnp.maximum(ts[0], ts[1]), jnp.maximum(ts[2], ts[3]))
            es = [jnp.exp(tk - m) for tk in ts]
            inv = 1.0 / (es[0] + es[1] + es[2] + es[3])
            att = [e * inv for e in es]
            x0 = rv[0] + att[0] * difs[0] + att[1] * difs[2] \
                + att[2] * difs[4] + att[3] * difs[6]
            x1 = rv[1] + att[0] * difs[1] + att[1] * difs[3] \
                + att[2] * difs[5] + att[3] * difs[7]
            nrm = jnp.cumsum(jnp.abs(x0) + jnp.abs(x1))
            pos = ch * _CHUNK + i
            plsc.store_scatter(norm_v, [jnp.full((_L,), pos, jnp.int32)], nrm,
                               mask=m15)
            for k in range(_K):
                plsc.store_scatter(
                    att_v, [jnp.full((_L,), k * _RPW + pos, jnp.int32)],
                    att[k], mask=m0)

        del row_body

    for k, ao in enumerate((a0_out, a1_out, a2_out, a3_out)):
        pltpu.sync_copy(att_v.at[pl.ds(k * _RPW, _RPW)],
                        ao.at[pl.ds(base_row, _RPW)])

    @pl.when(base_row < len_pos)
    def _write_pos():
        for j in range(3):
            pltpu.sync_copy(norm_v, pos_out.at[pl.ds(base_row + j * len_pos, _RPW)])

    @pl.when(base_row >= len_pos)
    def _write_neg():
        pltpu.sync_copy(norm_v, neg_out.at[pl.ds(base_row - len_pos, _RPW)])
        pltpu.sync_copy(y_v, y_out.at[pl.ds(base_row - len_pos, _RPW)])


def _sc_fused(ih, ir, it, entity_emb, relation_emb, fc1_w, fc1_b):
    b = _NW * _RPW
    dr = relation_emb.shape[1]
    mesh = plsc.VectorSubcoreMesh(core_axis_name="c", subcore_axis_name="s",
                                  num_cores=_NC, num_subcores=_NS)
    return pl.kernel(
        _fused_body,
        out_type=(
            jax.ShapeDtypeStruct((3 * (b // 4),), jnp.float32),
            jax.ShapeDtypeStruct((3 * (b // 4),), jnp.float32),
            jax.ShapeDtypeStruct((3 * (b // 4),), jnp.float32),
            jax.ShapeDtypeStruct((b,), jnp.float32),
            jax.ShapeDtypeStruct((b,), jnp.float32),
            jax.ShapeDtypeStruct((b,), jnp.float32),
            jax.ShapeDtypeStruct((b,), jnp.float32),
        ),
        mesh=mesh,
        scratch_types=[
            pltpu.VMEM((_RPW,), jnp.int32),
            pltpu.VMEM((_RPW,), jnp.int32),
            pltpu.VMEM((_RPW,), jnp.int32),
            pltpu.VMEM((_CHUNK, 128), jnp.float32),
            pltpu.VMEM((_CHUNK, 128), jnp.float32),
            pltpu.VMEM((_CHUNK, 128), jnp.float32),
            pltpu.VMEM((_CHUNK, 128), jnp.float32),
            pltpu.VMEM((_CHUNK, _ES), jnp.float32),
            pltpu.VMEM((_CHUNK, _ES), jnp.float32),
            pltpu.VMEM((_RPW,), jnp.float32),
            pltpu.VMEM((_RPW * _K,), jnp.float32),
            pltpu.VMEM((_RPW,), jnp.float32),
            pltpu.VMEM((1, 3 * _ES), jnp.float32),
            pltpu.VMEM((1,), jnp.float32),
            pltpu.SemaphoreType.DMA,
            pltpu.SemaphoreType.DMA,
            pltpu.SemaphoreType.DMA,
            pltpu.SemaphoreType.DMA,
            pltpu.SemaphoreType.DMA,
            pltpu.SemaphoreType.DMA,
        ],
        compiler_params=pltpu.CompilerParams(needs_layout_passes=False),
    )(ih, ir, it, entity_emb, relation_emb, fc1_w, fc1_b)


def kernel(batch_inputs, entity_emb, relation_emb, fc1_w, fc1_b):
    ih = batch_inputs[:, 0]
    ir = batch_inputs[:, 1]
    it = batch_inputs[:, 2]
    pos_norm, neg_norm, y, a0, a1, a2, a3 = _sc_fused(
        ih, ir, it, entity_emb, relation_emb, fc1_w, fc1_b)
    att = jnp.stack([a0, a1, a2, a3], axis=1)
    return (pos_norm, neg_norm, y, att)
